# baseline (speedup 1.0000x reference)
import functools

import jax
import jax.numpy as jnp
from jax.experimental import pallas as pl
from jax.experimental.pallas import tpu as pltpu

OUT_GROUPS = 1024
GROUPS_PER_BLK = 128


def _group_sums_body(n_cols, cb, nj, x_ref, s_ref, gs_ref, gsq_ref, np_ref):
    dn = (((1,), (0,)), ((), ()))
    norm = None
    for j in range(nj):
        c0 = j * cb
        c1 = min(c0 + cb, n_cols)
        xb = x_ref[:, c0:c1]
        xsq = xb * xb
        s = s_ref[...] if c1 - c0 == cb else s_ref[: c1 - c0, :]
        gs_ref[:, j * GROUPS_PER_BLK:(j + 1) * GROUPS_PER_BLK] = (
            jax.lax.dot_general(xb.astype(jnp.bfloat16), s, dn,
                                preferred_element_type=jnp.float32))
        gsq_ref[:, j * GROUPS_PER_BLK:(j + 1) * GROUPS_PER_BLK] = (
            jax.lax.dot_general(xsq.astype(jnp.bfloat16), s, dn,
                                preferred_element_type=jnp.float32))
        p = jnp.sum(xsq, axis=1, keepdims=True)
        norm = p if norm is None else norm + p
    np_ref[...] = jnp.broadcast_to(norm, np_ref.shape)


def _finalize_body(gs_ref, gsq_ref, np_ref, out_ref):
    gs = gs_ref[...]
    gsq = gsq_ref[...]
    norm = np_ref[:, :1]
    total = jnp.sum(gs, axis=1, keepdims=True)
    is_amp = jnp.all(jnp.abs(norm - 1.0) <= (1e-6 + 1e-5))
    out_ref[...] = jnp.where(is_amp, gsq, gs / total)


@jax.jit
def kernel(x):
    rows, n_cols = x.shape
    w = -(-n_cols // OUT_GROUPS)
    cb = w * GROUPS_PER_BLK
    nj = -(-OUT_GROUPS // GROUPS_PER_BLK)
    rb = min(64, rows)

    a = jax.lax.broadcasted_iota(jnp.int32, (cb, GROUPS_PER_BLK), 0)
    g = jax.lax.broadcasted_iota(jnp.int32, (cb, GROUPS_PER_BLK), 1)
    sel = ((a >= g * w) & (a < (g + 1) * w)).astype(jnp.bfloat16)

    gs, gsq, nparts = pl.pallas_call(
        functools.partial(_group_sums_body, n_cols, cb, nj),
        grid=(rows // rb,),
        in_specs=[
            pl.BlockSpec((rb, n_cols), lambda i: (i, 0)),
            pl.BlockSpec((cb, GROUPS_PER_BLK), lambda i: (0, 0)),
        ],
        out_specs=[
            pl.BlockSpec((rb, OUT_GROUPS), lambda i: (i, 0)),
            pl.BlockSpec((rb, OUT_GROUPS), lambda i: (i, 0)),
            pl.BlockSpec((rb, 128), lambda i: (i, 0)),
        ],
        out_shape=[
            jax.ShapeDtypeStruct((rows, OUT_GROUPS), jnp.float32),
            jax.ShapeDtypeStruct((rows, OUT_GROUPS), jnp.float32),
            jax.ShapeDtypeStruct((rows, 128), jnp.float32),
        ],
    )(x, sel)

    out = pl.pallas_call(
        _finalize_body,
        out_shape=jax.ShapeDtypeStruct((rows, OUT_GROUPS), jnp.float32),
    )(gs, gsq, nparts)
    return out
